# baseline (speedup 1.0000x reference)
import functools

import jax
import jax.numpy as jnp
from jax import lax
from jax.experimental import pallas as pl
from jax.experimental.pallas import tpu as pltpu
from jax.experimental.pallas import tpu_sc as plsc

_NC = 2
_NS = 16
_NW = _NC * _NS
_B = 80
_H = 16

_CHUNK = 624
_WB = 640


def _zero_acc(zero_hbm, acc, s, N):
    pltpu.sync_copy(zero_hbm.at[pl.ds(s * _CHUNK, _CHUNK)],
                    acc.at[pl.ds(s * _CHUNK, _CHUNK)])
    tail = N - _CHUNK * _NS
    if tail:
        @pl.when(s == _NS - 1)
        def _():
            pltpu.sync_copy(zero_hbm.at[pl.ds(_CHUNK * _NS, tail)],
                            acc.at[pl.ds(_CHUNK * _NS, tail)])


def _writeback_acc(acc, out_hbm, c, s, N, bounce, idxv, sem):
    base = s * _CHUNK
    tail = N - _CHUNK * _NS
    pltpu.sync_copy(acc.at[pl.ds(base, _CHUNK)], bounce.at[pl.ds(0, _CHUNK)])
    if tail:
        @pl.when(s == _NS - 1)
        def _():
            pltpu.sync_copy(acc.at[pl.ds(_CHUNK * _NS, tail)],
                            bounce.at[pl.ds(_CHUNK, tail)])
    cnt = jnp.where(s == _NS - 1, _CHUNK + tail, _CHUNK)
    cbase = c * 8 * N + base * 8

    def mkidx(j, carry):
        k = j * _H + lax.iota(jnp.int32, _H)
        idxv[pl.ds(j * _H, _H)] = jnp.where(k < cnt, cbase + k * 8, cbase + 1)
        return carry
    lax.fori_loop(0, _WB // _H, mkidx, 0)

    for off in range(0, _WB, 128):
        pltpu.async_copy(bounce.at[pl.ds(off, 128)],
                         out_hbm.at[idxv.at[pl.ds(off, 128)]], sem)
    for off in range(0, _WB, 128):
        pltpu.make_async_copy(bounce.at[pl.ds(0, 128)],
                              out_hbm.at[idxv.at[pl.ds(0, 128)]], sem).wait()


def _make_deg_kernel(N, E):
    ew = E // _NW
    nb = ew // _B
    mesh = plsc.VectorSubcoreMesh(core_axis_name="c", subcore_axis_name="s")

    @functools.partial(
        pl.kernel,
        out_type=jax.ShapeDtypeStruct((_NC * 128 * N,), jnp.float32),
        mesh=mesh,
        compiler_params=pltpu.CompilerParams(use_tc_tiling_on_sc=False),
        scratch_types=[
            pltpu.VMEM((ew,), jnp.int32),
            pltpu.VMEM((_B,), jnp.float32),
            pltpu.VMEM((_WB,), jnp.float32),
            pltpu.VMEM((_WB,), jnp.int32),
            pltpu.VMEM_SHARED((N,), jnp.float32),
            pltpu.SemaphoreType.DMA,
        ],
    )
    def k(edge_hbm, zero_hbm, out_hbm, dstv, ones_v, bounce, idxv, acc, sem):
        c = lax.axis_index("c")
        s = lax.axis_index("s")
        w = c * _NS + s
        pltpu.sync_copy(zero_hbm.at[pl.ds(s * _CHUNK, _CHUNK)],
                        acc.at[pl.ds(s * _CHUNK, _CHUNK)])
        tail = N - _CHUNK * _NS
        if tail:
            @pl.when(s == _NS - 1)
            def _():
                pltpu.sync_copy(zero_hbm.at[pl.ds(_CHUNK * _NS, tail)],
                                acc.at[pl.ds(_CHUNK * _NS, tail)])
        pltpu.sync_copy(edge_hbm.at[1, pl.ds(w * ew, ew)], dstv)

        def mkones(i, carry):
            ones_v[pl.ds(i * _H, _H)] = jnp.full((_H,), 1.0, jnp.float32)
            return carry
        lax.fori_loop(0, _B // _H, mkones, 0)
        plsc.subcore_barrier()

        def fire(j, carry):
            pltpu.async_copy(ones_v, acc.at[dstv.at[pl.ds(j * _B, _B)]],
                             sem, add=True)
            return carry
        lax.fori_loop(0, nb, fire, 0)

        def drain(j, carry):
            pltpu.make_async_copy(ones_v, acc.at[dstv.at[pl.ds(0, _B)]],
                                  sem).wait()
            return carry
        lax.fori_loop(0, nb, drain, 0)
        plsc.subcore_barrier()

        base = s * _CHUNK
        pltpu.sync_copy(acc.at[pl.ds(base, _CHUNK)], bounce.at[pl.ds(0, _CHUNK)])
        if tail:
            @pl.when(s == _NS - 1)
            def _():
                pltpu.sync_copy(acc.at[pl.ds(_CHUNK * _NS, tail)],
                                bounce.at[pl.ds(_CHUNK, tail)])
        cnt = jnp.where(s == _NS - 1, _CHUNK + tail, _CHUNK)
        cbase = c * 128 * N + base * 128

        def mkidx(j, carry):
            kk = j * _H + lax.iota(jnp.int32, _H)
            idxv[pl.ds(j * _H, _H)] = jnp.where(kk < cnt, cbase + kk * 128,
                                                cbase + 1)
            return carry
        lax.fori_loop(0, _WB // _H, mkidx, 0)

        for off in range(0, _WB, 128):
            pltpu.async_copy(bounce.at[pl.ds(off, 128)],
                             out_hbm.at[idxv.at[pl.ds(off, 128)]], sem)
        for off in range(0, _WB, 128):
            pltpu.make_async_copy(bounce.at[pl.ds(0, 128)],
                                  out_hbm.at[idxv.at[pl.ds(0, 128)]],
                                  sem).wait()

    return k


_NP = 5


def _make_msg_kernel(N, E):
    ew = E // _NW
    nb = ew // _B
    pb = nb // _NP
    mesh = plsc.VectorSubcoreMesh(core_axis_name="c", subcore_axis_name="s")

    @functools.partial(
        pl.kernel,
        out_type=jax.ShapeDtypeStruct((_NC * 8 * N, _H), jnp.float32),
        mesh=mesh,
        compiler_params=pltpu.CompilerParams(use_tc_tiling_on_sc=False),
        scratch_types=[
            pltpu.VMEM((ew,), jnp.int32),
            pltpu.VMEM((ew,), jnp.int32),
            pltpu.VMEM((2, pb * _B, _H), jnp.float32),
            pltpu.VMEM((_WB, _H), jnp.float32),
            pltpu.VMEM((_WB,), jnp.int32),
            pltpu.VMEM_SHARED((N, _H), jnp.float32),
            pltpu.SemaphoreType.DMA,
            pltpu.SemaphoreType.DMA,
        ],
    )
    def k(g_hbm, edge_hbm, zero_hbm, out_hbm,
          srcv, dstv, rows, bounce, idxv, acc, gsem, ssem):
        c = lax.axis_index("c")
        s = lax.axis_index("s")
        w = c * _NS + s
        _zero_acc(zero_hbm, acc, s, N)
        pltpu.sync_copy(edge_hbm.at[0, pl.ds(w * ew, ew)], srcv)
        pltpu.sync_copy(edge_hbm.at[1, pl.ds(w * ew, ew)], dstv)
        plsc.subcore_barrier()

        def fire_gathers(p, half):
            def f(j, carry):
                base = p * pb * _B + j * _B
                pltpu.async_copy(g_hbm.at[srcv.at[pl.ds(base, _B)]],
                                 rows.at[half, pl.ds(j * _B, _B)], gsem)
                return carry
            lax.fori_loop(0, pb, f, 0)

        def fire_scatters(p, half):
            def f(j, carry):
                base = p * pb * _B + j * _B
                pltpu.async_copy(rows.at[half, pl.ds(j * _B, _B)],
                                 acc.at[dstv.at[pl.ds(base, _B)]],
                                 ssem, add=True)
                return carry
            lax.fori_loop(0, pb, f, 0)

        def drain_g(_):
            def f(j, carry):
                pltpu.make_async_copy(g_hbm.at[srcv.at[pl.ds(0, _B)]],
                                      rows.at[0, pl.ds(0, _B)], gsem).wait()
                return carry
            lax.fori_loop(0, pb, f, 0)

        def drain_s(_):
            def f(j, carry):
                pltpu.make_async_copy(rows.at[0, pl.ds(0, _B)],
                                      acc.at[dstv.at[pl.ds(0, _B)]],
                                      ssem).wait()
                return carry
            lax.fori_loop(0, pb, f, 0)

        fire_gathers(0, 0)
        for p in range(_NP):
            h = p % 2
            drain_g(None)
            if p >= 1:
                drain_s(None)
            if p + 1 < _NP:
                fire_gathers(p + 1, 1 - h)
            fire_scatters(p, h)
        drain_s(None)

        plsc.subcore_barrier()
        _writeback_acc(acc, out_hbm, c, s, N, bounce, idxv, ssem)

    return k


_BM = 2000


def _tc1(x, W1, degp128):
    N, D = x.shape
    H = W1.shape[1]

    def body(x_ref, w_ref, dp_ref, g_ref, dinv_ref):
        deg = dp_ref[0, :, :1] + dp_ref[1, :, :1] + 1.0
        dinv = lax.rsqrt(deg)
        a = jnp.dot(x_ref[...], w_ref[...], preferred_element_type=jnp.float32)
        g_ref[:, : _H] = a * dinv
        dinv_ref[:, : _H] = jnp.broadcast_to(dinv, (_BM, H))

    return pl.pallas_call(
        body,
        grid=(N // _BM,),
        in_specs=[pl.BlockSpec((_BM, D), lambda i: (i, 0)),
                  pl.BlockSpec((D, H), lambda i: (0, 0)),
                  pl.BlockSpec((_NC, _BM, 128), lambda i: (0, i, 0))],
        out_specs=[pl.BlockSpec((_BM, 128), lambda i: (i, 0))] * 2,
        out_shape=[jax.ShapeDtypeStruct((N, 128), jnp.float32)] * 2,
    )(x, W1, degp128)


def _tc2(mp128, g128, dinv128, b, W2):
    N = g128.shape[0]
    H = _H

    def body(m_ref, g_ref, dinv_ref, b_ref, w_ref, out_ref):
        dinv = dinv_ref[:, : _H]
        pre = dinv * (m_ref[0, :, : _H] + m_ref[1, :, : _H] + g_ref[:, : _H])
        h = jnp.maximum(pre + b_ref[...], 0.0)
        a = jnp.dot(h, w_ref[...], preferred_element_type=jnp.float32)
        out_ref[:, : _H] = a * dinv

    return pl.pallas_call(
        body,
        grid=(N // _BM,),
        in_specs=[pl.BlockSpec((_NC, _BM, 128), lambda i: (0, i, 0)),
                  pl.BlockSpec((_BM, 128), lambda i: (i, 0)),
                  pl.BlockSpec((_BM, 128), lambda i: (i, 0)),
                  pl.BlockSpec((1, H), lambda i: (0, 0)),
                  pl.BlockSpec((H, H), lambda i: (0, 0))],
        out_specs=pl.BlockSpec((_BM, 128), lambda i: (i, 0)),
        out_shape=jax.ShapeDtypeStruct((N, 128), jnp.float32),
    )(mp128, g128, dinv128, b, W2)


def _tc3(mp128, g128, dinv128, b, Wfc, bfc):
    N = g128.shape[0]
    H = _H
    OUT = Wfc.shape[1]

    def body(m_ref, g_ref, dinv_ref, b_ref, w_ref, bfc_ref, out_ref):
        dinv = dinv_ref[:, : _H]
        pre = dinv * (m_ref[0, :, : _H] + m_ref[1, :, : _H] + g_ref[:, : _H])
        h = jnp.maximum(pre + b_ref[...], 0.0)
        out_ref[...] = (jnp.dot(h, w_ref[...], preferred_element_type=jnp.float32)
                        + bfc_ref[...])

    return pl.pallas_call(
        body,
        grid=(N // _BM,),
        in_specs=[pl.BlockSpec((_NC, _BM, 128), lambda i: (0, i, 0)),
                  pl.BlockSpec((_BM, 128), lambda i: (i, 0)),
                  pl.BlockSpec((_BM, 128), lambda i: (i, 0)),
                  pl.BlockSpec((1, H), lambda i: (0, 0)),
                  pl.BlockSpec((H, OUT), lambda i: (0, 0)),
                  pl.BlockSpec((1, OUT), lambda i: (0, 0))],
        out_specs=pl.BlockSpec((_BM, OUT), lambda i: (i, 0)),
        out_shape=jax.ShapeDtypeStruct((N, OUT), jnp.float32),
    )(mp128, g128, dinv128, b, Wfc, bfc)


def kernel(x, edge_index, output_size, W1, b1, W2, b2, Wfc, bfc):
    N, D = x.shape
    E = edge_index.shape[1]
    H = W1.shape[1]
    OUT = Wfc.shape[1]
    assert H == _H and E % (_NW * _B) == 0 and N % 8 == 0

    edge8 = edge_index.astype(jnp.int32) * jnp.array([[8], [1]], jnp.int32)
    zeros = jnp.zeros((N, H), jnp.float32)
    zeros1 = jnp.zeros((N,), jnp.float32)

    deg_k = _make_deg_kernel(N, E)
    msg_k = _make_msg_kernel(N, E)

    degp = deg_k(edge8, zeros1)
    g1, dinv128 = _tc1(x, W1, degp.reshape(_NC, N, 128))
    m1 = msg_k(g1.reshape(8 * N, H), edge8, zeros)
    g2 = _tc2(m1.reshape(_NC, N, 128), g1, dinv128, b1.reshape(1, H), W2)
    m2 = msg_k(g2.reshape(8 * N, H), edge8, zeros)
    return _tc3(m2.reshape(_NC, N, 128), g2, dinv128, b2.reshape(1, H), Wfc,
                bfc.reshape(1, OUT))


# speedup vs baseline: 1.1545x; 1.1545x over previous
import functools

import jax
import jax.numpy as jnp
from jax import lax
from jax.experimental import pallas as pl
from jax.experimental.pallas import tpu as pltpu
from jax.experimental.pallas import tpu_sc as plsc

_NC = 2
_NS = 16
_NW = _NC * _NS
_B = 80
_H = 16

_CHUNK = 624
_WB = 640


def _zero_acc(zero_hbm, acc, s, N):
    pltpu.sync_copy(zero_hbm.at[pl.ds(s * _CHUNK, _CHUNK)],
                    acc.at[pl.ds(s * _CHUNK, _CHUNK)])
    tail = N - _CHUNK * _NS
    if tail:
        @pl.when(s == _NS - 1)
        def _():
            pltpu.sync_copy(zero_hbm.at[pl.ds(_CHUNK * _NS, tail)],
                            acc.at[pl.ds(_CHUNK * _NS, tail)])


def _writeback_acc(acc, out_hbm, c, s, N, bounce, idxv, sem):
    base = s * _CHUNK
    tail = N - _CHUNK * _NS
    pltpu.sync_copy(acc.at[pl.ds(base, _CHUNK)], bounce.at[pl.ds(0, _CHUNK)])
    if tail:
        @pl.when(s == _NS - 1)
        def _():
            pltpu.sync_copy(acc.at[pl.ds(_CHUNK * _NS, tail)],
                            bounce.at[pl.ds(_CHUNK, tail)])
    cnt = jnp.where(s == _NS - 1, _CHUNK + tail, _CHUNK)
    cbase = c * 8 * N + base * 8

    def mkidx(j, carry):
        k = j * _H + lax.iota(jnp.int32, _H)
        idxv[pl.ds(j * _H, _H)] = jnp.where(k < cnt, cbase + k * 8, cbase + 1)
        return carry
    lax.fori_loop(0, _WB // _H, mkidx, 0)

    for off in range(0, _WB, 128):
        pltpu.async_copy(bounce.at[pl.ds(off, 128)],
                         out_hbm.at[idxv.at[pl.ds(off, 128)]], sem)
    for off in range(0, _WB, 128):
        pltpu.make_async_copy(bounce.at[pl.ds(0, 128)],
                              out_hbm.at[idxv.at[pl.ds(0, 128)]], sem).wait()


def _make_deg_kernel(N, E):
    ew = E // _NW
    nb = ew // _B
    mesh = plsc.VectorSubcoreMesh(core_axis_name="c", subcore_axis_name="s")

    @functools.partial(
        pl.kernel,
        out_type=jax.ShapeDtypeStruct((_NC * 8 * N, _H), jnp.float32),
        mesh=mesh,
        compiler_params=pltpu.CompilerParams(use_tc_tiling_on_sc=False),
        scratch_types=[
            pltpu.VMEM((ew,), jnp.int32),
            pltpu.VMEM((_B, _H), jnp.float32),
            pltpu.VMEM((_WB, _H), jnp.float32),
            pltpu.VMEM((_WB,), jnp.int32),
            pltpu.VMEM_SHARED((N, _H), jnp.float32),
            pltpu.SemaphoreType.DMA,
        ],
    )
    def k(edge_hbm, zero_hbm, out_hbm, dstv, ones_v, bounce, idxv, acc, sem):
        c = lax.axis_index("c")
        s = lax.axis_index("s")
        w = c * _NS + s
        _zero_acc(zero_hbm, acc, s, N)
        pltpu.sync_copy(edge_hbm.at[1, pl.ds(w * ew, ew)], dstv)

        def mkones(i, carry):
            ones_v[i, :] = jnp.full((_H,), 1.0, jnp.float32)
            return carry
        lax.fori_loop(0, _B, mkones, 0)
        plsc.subcore_barrier()

        def fire(j, carry):
            pltpu.async_copy(ones_v, acc.at[dstv.at[pl.ds(j * _B, _B)]],
                             sem, add=True)
            return carry
        lax.fori_loop(0, nb, fire, 0)

        def drain(j, carry):
            pltpu.make_async_copy(ones_v, acc.at[dstv.at[pl.ds(0, _B)]],
                                  sem).wait()
            return carry
        lax.fori_loop(0, nb, drain, 0)
        plsc.subcore_barrier()
        _writeback_acc(acc, out_hbm, c, s, N, bounce, idxv, sem)

    return k


_NP = 5


def _make_msg_kernel(N, E):
    ew = E // _NW
    nb = ew // _B
    pb = nb // _NP
    mesh = plsc.VectorSubcoreMesh(core_axis_name="c", subcore_axis_name="s")

    @functools.partial(
        pl.kernel,
        out_type=jax.ShapeDtypeStruct((_NC * 8 * N, _H), jnp.float32),
        mesh=mesh,
        compiler_params=pltpu.CompilerParams(use_tc_tiling_on_sc=False),
        scratch_types=[
            pltpu.VMEM((ew,), jnp.int32),
            pltpu.VMEM((ew,), jnp.int32),
            pltpu.VMEM((2, pb * _B, _H), jnp.float32),
            pltpu.VMEM((_WB, _H), jnp.float32),
            pltpu.VMEM((_WB,), jnp.int32),
            pltpu.VMEM_SHARED((N, _H), jnp.float32),
            pltpu.SemaphoreType.DMA,
            pltpu.SemaphoreType.DMA,
        ],
    )
    def k(g_hbm, edge_hbm, zero_hbm, out_hbm,
          srcv, dstv, rows, bounce, idxv, acc, gsem, ssem):
        c = lax.axis_index("c")
        s = lax.axis_index("s")
        w = c * _NS + s
        _zero_acc(zero_hbm, acc, s, N)
        pltpu.sync_copy(edge_hbm.at[0, pl.ds(w * ew, ew)], srcv)
        pltpu.sync_copy(edge_hbm.at[1, pl.ds(w * ew, ew)], dstv)
        plsc.subcore_barrier()

        def fire_gathers(p, half):
            def f(j, carry):
                base = p * pb * _B + j * _B
                pltpu.async_copy(g_hbm.at[srcv.at[pl.ds(base, _B)]],
                                 rows.at[half, pl.ds(j * _B, _B)], gsem)
                return carry
            lax.fori_loop(0, pb, f, 0)

        def fire_scatters(p, half):
            def f(j, carry):
                base = p * pb * _B + j * _B
                pltpu.async_copy(rows.at[half, pl.ds(j * _B, _B)],
                                 acc.at[dstv.at[pl.ds(base, _B)]],
                                 ssem, add=True)
                return carry
            lax.fori_loop(0, pb, f, 0)

        def drain_g(_):
            def f(j, carry):
                pltpu.make_async_copy(g_hbm.at[srcv.at[pl.ds(0, _B)]],
                                      rows.at[0, pl.ds(0, _B)], gsem).wait()
                return carry
            lax.fori_loop(0, pb, f, 0)

        def drain_s(_):
            def f(j, carry):
                pltpu.make_async_copy(rows.at[0, pl.ds(0, _B)],
                                      acc.at[dstv.at[pl.ds(0, _B)]],
                                      ssem).wait()
                return carry
            lax.fori_loop(0, pb, f, 0)

        fire_gathers(0, 0)
        for p in range(_NP):
            h = p % 2
            drain_g(None)
            if p >= 1:
                drain_s(None)
            if p + 1 < _NP:
                fire_gathers(p + 1, 1 - h)
            fire_scatters(p, h)
        drain_s(None)

        plsc.subcore_barrier()
        _writeback_acc(acc, out_hbm, c, s, N, bounce, idxv, ssem)

    return k


def _tc1a(x, W1):
    N, D = x.shape
    H = W1.shape[1]

    def body(x_ref, w_ref, a_ref):
        a_ref[:, : _H] = jnp.dot(x_ref[...], w_ref[...],
                                 preferred_element_type=jnp.float32)

    return pl.pallas_call(
        body,
        out_shape=jax.ShapeDtypeStruct((N, 128), jnp.float32),
    )(x, W1)


def _tc1b(a128, degp128):
    N = a128.shape[0]

    def body(a_ref, dp_ref, g_ref, dinv_ref):
        deg = dp_ref[0, :, : _H] + dp_ref[1, :, : _H] + 1.0
        dinv = lax.rsqrt(deg)
        g_ref[:, : _H] = a_ref[:, : _H] * dinv
        dinv_ref[:, : _H] = dinv

    return pl.pallas_call(
        body,
        out_shape=[jax.ShapeDtypeStruct((N, 128), jnp.float32)] * 2,
    )(a128, degp128)


def _tc2(mp128, g128, dinv128, b, W2):
    N = g128.shape[0]

    def body(m_ref, g_ref, dinv_ref, b_ref, w_ref, out_ref):
        dinv = dinv_ref[:, : _H]
        pre = dinv * (m_ref[0, :, : _H] + m_ref[1, :, : _H] + g_ref[:, : _H])
        h = jnp.maximum(pre + b_ref[...], 0.0)
        a = jnp.dot(h, w_ref[...], preferred_element_type=jnp.float32)
        out_ref[:, : _H] = a * dinv

    return pl.pallas_call(
        body,
        out_shape=jax.ShapeDtypeStruct((N, 128), jnp.float32),
    )(mp128, g128, dinv128, b, W2)


def _tc3(mp128, g128, dinv128, b, Wfc, bfc):
    N = g128.shape[0]
    OUT = Wfc.shape[1]

    def body(m_ref, g_ref, dinv_ref, b_ref, w_ref, bfc_ref, out_ref):
        dinv = dinv_ref[:, : _H]
        pre = dinv * (m_ref[0, :, : _H] + m_ref[1, :, : _H] + g_ref[:, : _H])
        h = jnp.maximum(pre + b_ref[...], 0.0)
        out_ref[...] = (jnp.dot(h, w_ref[...], preferred_element_type=jnp.float32)
                        + bfc_ref[...])

    return pl.pallas_call(
        body,
        out_shape=jax.ShapeDtypeStruct((N, OUT), jnp.float32),
    )(mp128, g128, dinv128, b, Wfc, bfc)


def kernel(x, edge_index, output_size, W1, b1, W2, b2, Wfc, bfc):
    N, D = x.shape
    E = edge_index.shape[1]
    H = W1.shape[1]
    OUT = Wfc.shape[1]
    assert H == _H and E % (_NW * _B) == 0 and N % 8 == 0

    edge8 = edge_index.astype(jnp.int32) * jnp.array([[8], [1]], jnp.int32)
    zeros = jnp.zeros((N, H), jnp.float32)

    deg_k = _make_deg_kernel(N, E)
    msg_k = _make_msg_kernel(N, E)

    degp = deg_k(edge8, zeros)
    a1 = _tc1a(x, W1)
    g1, dinv128 = _tc1b(a1, degp.reshape(_NC, N, 128))
    m1 = msg_k(g1.reshape(8 * N, H), edge8, zeros)
    g2 = _tc2(m1.reshape(_NC, N, 128), g1, dinv128, b1.reshape(1, H), W2)
    m2 = msg_k(g2.reshape(8 * N, H), edge8, zeros)
    return _tc3(m2.reshape(_NC, N, 128), g2, dinv128, b2.reshape(1, H), Wfc,
                bfc.reshape(1, OUT))


# speedup vs baseline: 1.2559x; 1.0878x over previous
import functools

import jax
import jax.numpy as jnp
from jax import lax
from jax.experimental import pallas as pl
from jax.experimental.pallas import tpu as pltpu
from jax.experimental.pallas import tpu_sc as plsc

_NC = 2
_NS = 16
_NW = _NC * _NS
_B = 80
_H = 16

_CHUNK = 624


def _zero_acc(zero_hbm, acc, s, N):
    pltpu.sync_copy(zero_hbm.at[pl.ds(s * _CHUNK, _CHUNK)],
                    acc.at[pl.ds(s * _CHUNK, _CHUNK)])
    tail = N - _CHUNK * _NS
    if tail:
        @pl.when(s == _NS - 1)
        def _():
            pltpu.sync_copy(zero_hbm.at[pl.ds(_CHUNK * _NS, tail)],
                            acc.at[pl.ds(_CHUNK * _NS, tail)])


def _writeback_acc(acc, out_hbm, c, s, N):
    base = s * _CHUNK
    tail = N - _CHUNK * _NS
    pltpu.sync_copy(acc.at[pl.ds(base, _CHUNK)],
                    out_hbm.at[pl.ds(c * N + base, _CHUNK)])
    if tail:
        @pl.when(s == _NS - 1)
        def _():
            pltpu.sync_copy(acc.at[pl.ds(_CHUNK * _NS, tail)],
                            out_hbm.at[pl.ds(c * N + _CHUNK * _NS, tail)])


def _make_deg_kernel(N, E):
    ew = E // _NW
    nb = ew // _B
    mesh = plsc.VectorSubcoreMesh(core_axis_name="c", subcore_axis_name="s")

    @functools.partial(
        pl.kernel,
        out_type=jax.ShapeDtypeStruct((_NC * N, _H), jnp.float32),
        mesh=mesh,
        compiler_params=pltpu.CompilerParams(use_tc_tiling_on_sc=False),
        scratch_types=[
            pltpu.VMEM((ew,), jnp.int32),
            pltpu.VMEM((_B, _H), jnp.float32),
            pltpu.VMEM_SHARED((N, _H), jnp.float32),
            pltpu.SemaphoreType.DMA,
        ],
    )
    def k(edge_hbm, zero_hbm, out_hbm, dstv, ones_v, acc, sem):
        c = lax.axis_index("c")
        s = lax.axis_index("s")
        w = c * _NS + s
        _zero_acc(zero_hbm, acc, s, N)
        pltpu.sync_copy(edge_hbm.at[1, pl.ds(w * ew, ew)], dstv)

        def mkones(i, carry):
            ones_v[i, :] = jnp.full((_H,), 1.0, jnp.float32)
            return carry
        lax.fori_loop(0, _B, mkones, 0)
        plsc.subcore_barrier()

        def fire(j, carry):
            pltpu.async_copy(ones_v, acc.at[dstv.at[pl.ds(j * _B, _B)]],
                             sem, add=True)
            return carry
        lax.fori_loop(0, nb, fire, 0)

        def drain(j, carry):
            pltpu.make_async_copy(ones_v, acc.at[dstv.at[pl.ds(0, _B)]],
                                  sem).wait()
            return carry
        lax.fori_loop(0, nb, drain, 0)
        plsc.subcore_barrier()
        _writeback_acc(acc, out_hbm, c, s, N)

    return k


_NP = 5


def _make_msg_kernel(N, E):
    ew = E // _NW
    nb = ew // _B
    pb = nb // _NP
    mesh = plsc.VectorSubcoreMesh(core_axis_name="c", subcore_axis_name="s")

    @functools.partial(
        pl.kernel,
        out_type=jax.ShapeDtypeStruct((_NC * N, _H), jnp.float32),
        mesh=mesh,
        compiler_params=pltpu.CompilerParams(use_tc_tiling_on_sc=False),
        scratch_types=[
            pltpu.VMEM((ew,), jnp.int32),
            pltpu.VMEM((ew,), jnp.int32),
            pltpu.VMEM((2, pb * _B, _H), jnp.float32),
            pltpu.VMEM_SHARED((N, _H), jnp.float32),
            pltpu.SemaphoreType.DMA,
            pltpu.SemaphoreType.DMA,
        ],
    )
    def k(g_hbm, edge_hbm, zero_hbm, out_hbm,
          srcv, dstv, rows, acc, gsem, ssem):
        c = lax.axis_index("c")
        s = lax.axis_index("s")
        w = c * _NS + s
        _zero_acc(zero_hbm, acc, s, N)
        pltpu.sync_copy(edge_hbm.at[0, pl.ds(w * ew, ew)], srcv)
        pltpu.sync_copy(edge_hbm.at[1, pl.ds(w * ew, ew)], dstv)
        plsc.subcore_barrier()

        def fire_gathers(p, half):
            def f(j, carry):
                base = p * pb * _B + j * _B
                pltpu.async_copy(g_hbm.at[srcv.at[pl.ds(base, _B)]],
                                 rows.at[half, pl.ds(j * _B, _B)], gsem)
                return carry
            lax.fori_loop(0, pb, f, 0)

        def fire_scatters(p, half):
            def f(j, carry):
                base = p * pb * _B + j * _B
                pltpu.async_copy(rows.at[half, pl.ds(j * _B, _B)],
                                 acc.at[dstv.at[pl.ds(base, _B)]],
                                 ssem, add=True)
                return carry
            lax.fori_loop(0, pb, f, 0)

        def drain_g(_):
            def f(j, carry):
                pltpu.make_async_copy(g_hbm.at[srcv.at[pl.ds(0, _B)]],
                                      rows.at[0, pl.ds(0, _B)], gsem).wait()
                return carry
            lax.fori_loop(0, pb, f, 0)

        def drain_s(_):
            def f(j, carry):
                pltpu.make_async_copy(rows.at[0, pl.ds(0, _B)],
                                      acc.at[dstv.at[pl.ds(0, _B)]],
                                      ssem).wait()
                return carry
            lax.fori_loop(0, pb, f, 0)

        fire_gathers(0, 0)
        for p in range(_NP):
            h = p % 2
            drain_g(None)
            if p >= 1:
                drain_s(None)
            if p + 1 < _NP:
                fire_gathers(p + 1, 1 - h)
            fire_scatters(p, h)
        drain_s(None)

        plsc.subcore_barrier()
        _writeback_acc(acc, out_hbm, c, s, N)

    return k


def _tc1(x1024, W1B, degpc):
    NR = x1024.shape[0]

    def body(x_ref, w_ref, dp_ref, g_ref, dinv_ref):
        deg = dp_ref[0] + dp_ref[1] + 1.0
        dinv = lax.rsqrt(deg)
        a = jnp.dot(x_ref[...], w_ref[...], preferred_element_type=jnp.float32)
        g_ref[...] = a * dinv
        dinv_ref[...] = dinv

    return pl.pallas_call(
        body,
        out_shape=[jax.ShapeDtypeStruct((NR, 128), jnp.float32)] * 2,
    )(x1024, W1B, degpc)


def _tc2(mpc, gc, dinvc, b128, W2B):
    NR = gc.shape[0]

    def body(m_ref, g_ref, dinv_ref, b_ref, w_ref, out_ref):
        dinv = dinv_ref[...]
        pre = dinv * (m_ref[0] + m_ref[1] + g_ref[...])
        h = jnp.maximum(pre + b_ref[...], 0.0)
        a = jnp.dot(h, w_ref[...], preferred_element_type=jnp.float32)
        out_ref[...] = a * dinv

    return pl.pallas_call(
        body,
        out_shape=jax.ShapeDtypeStruct((NR, 128), jnp.float32),
    )(mpc, gc, dinvc, b128, W2B)


def _tc3(mpc, gc, dinvc, b128, WfcB, bfcB):
    NR = gc.shape[0]
    OUTB = WfcB.shape[1]

    def body(m_ref, g_ref, dinv_ref, b_ref, w_ref, bfc_ref, out_ref):
        pre = dinv_ref[...] * (m_ref[0] + m_ref[1] + g_ref[...])
        h = jnp.maximum(pre + b_ref[...], 0.0)
        out_ref[...] = (jnp.dot(h, w_ref[...], preferred_element_type=jnp.float32)
                        + bfc_ref[...])

    return pl.pallas_call(
        body,
        out_shape=jax.ShapeDtypeStruct((NR, OUTB), jnp.float32),
    )(mpc, gc, dinvc, b128, WfcB, bfcB)


def kernel(x, edge_index, output_size, W1, b1, W2, b2, Wfc, bfc):
    N, D = x.shape
    E = edge_index.shape[1]
    H = W1.shape[1]
    OUT = Wfc.shape[1]
    assert H == _H and E % (_NW * _B) == 0 and N % 8 == 0
    NR = N // 8

    edge32 = edge_index.astype(jnp.int32)
    zeros = jnp.zeros((N, H), jnp.float32)
    eye8 = jnp.eye(8, dtype=jnp.float32)
    x1024 = x.reshape(NR, 8 * D)
    W1B = jnp.kron(eye8, W1)
    W2B = jnp.kron(eye8, W2)
    WfcB = jnp.kron(eye8, Wfc)
    b128 = jnp.tile(b1, 8).reshape(1, 128)
    b228 = jnp.tile(b2, 8).reshape(1, 128)
    bfcB = jnp.tile(bfc, 8).reshape(1, 8 * OUT)

    deg_k = _make_deg_kernel(N, E)
    msg_k = _make_msg_kernel(N, E)

    degp = deg_k(edge32, zeros)
    g1, dinvc = _tc1(x1024, W1B, degp.reshape(_NC, NR, 128))
    m1 = msg_k(g1.reshape(N, H), edge32, zeros)
    g2 = _tc2(m1.reshape(_NC, NR, 128), g1, dinvc, b128, W2B)
    m2 = msg_k(g2.reshape(N, H), edge32, zeros)
    out8 = _tc3(m2.reshape(_NC, NR, 128), g2, dinvc, b228, WfcB, bfcB)
    return out8.reshape(N, OUT)
